# baseline (speedup 1.0000x reference)
import functools

import jax
import jax.numpy as jnp
from jax import lax
from jax.experimental import pallas as pl
from jax.experimental.pallas import tpu as pltpu
from jax.experimental.pallas import tpu_sc as plsc

N = 10000
E = 320000
IN_C = 128
HID = 64
OUT = 64

NC = 2
NS = 16
NW = NC * NS
EPW = E // NW
K = 80
GS = 2
NCH = EPW // K
NG = (NCH - 1) // GS

DGC = 624
DGW = 640
ZR = 32
ONES = 80

_f32 = jnp.float32


@functools.cache
def _sc_scatter_pass(with_deg: bool):
    out_type = [jax.ShapeDtypeStruct((NC, N, IN_C), _f32)]
    if with_deg:
        out_type.append(jax.ShapeDtypeStruct((NC * N,), _f32))

    scratch = [
        pltpu.VMEM_SHARED((N, IN_C), _f32),
        pltpu.VMEM((ZR, IN_C), _f32),
        [[pltpu.VMEM((K, IN_C), _f32) for _ in range(GS)] for _ in range(2)],
        [[pltpu.VMEM((K,), jnp.int32) for _ in range(GS)] for _ in range(2)],
        [[pltpu.VMEM((K,), jnp.int32) for _ in range(GS)] for _ in range(2)],
        [[pltpu.SemaphoreType.DMA for _ in range(GS)] for _ in range(2)],
        [[pltpu.SemaphoreType.DMA for _ in range(GS)] for _ in range(2)],
        [[pltpu.SemaphoreType.DMA for _ in range(GS)] for _ in range(2)],
    ]
    if with_deg:
        scratch += [
            pltpu.VMEM_SHARED((N,), _f32),
            pltpu.VMEM((DGW,), _f32),
            pltpu.VMEM((ONES,), _f32),
            [[pltpu.SemaphoreType.DMA for _ in range(GS)] for _ in range(2)],
        ]

    def body(table_hbm, edge_hbm, *rest):
        if with_deg:
            (agg_out, deg_out,
             acc_sh, zero_buf, rows, srcb, dstb, isem, gsem, ssem,
             deg_sh, zdeg_buf, ones_buf, dsem) = rest
        else:
            (agg_out,
             acc_sh, zero_buf, rows, srcb, dstb, isem, gsem, ssem) = rest
            deg_out = deg_sh = zdeg_buf = ones_buf = dsem = None

        cid = lax.axis_index("c")
        sid = lax.axis_index("s")
        base_e = (cid * NS + sid) * EPW

        zv = jnp.zeros((16,), _f32)

        def zfill(i, _):
            zero_buf[i // (IN_C // 16), pl.ds((i % (IN_C // 16)) * 16, 16)] = zv
            return 0
        lax.fori_loop(0, ZR * (IN_C // 16), zfill, 0)
        if with_deg:
            def zfill_d(i, _):
                zdeg_buf[pl.ds(i * 16, 16)] = zv
                return 0
            lax.fori_loop(0, DGW // 16, zfill_d, 0)
            ov = jnp.ones((16,), _f32)
            for j in range(ONES // 16):
                ones_buf[pl.ds(j * 16, 16)] = ov

        for j in range(DGW // ZR):
            pltpu.sync_copy(zero_buf, acc_sh.at[pl.ds(sid * DGC + j * ZR, ZR)])
        if with_deg:
            pltpu.sync_copy(zdeg_buf, deg_sh.at[pl.ds(sid * DGC, DGW)])
        plsc.subcore_barrier()

        def fire_gathers(g, s):
            ih = []
            for j in range(GS):
                e0 = base_e + (g * GS + j) * K
                ih.append((
                    pltpu.async_copy(edge_hbm.at[pl.ds(e0, K)], srcb[s][j], isem[s][j]),
                    pltpu.async_copy(edge_hbm.at[pl.ds(E + e0, K)], dstb[s][j], isem[s][j]),
                ))
            for j in range(GS):
                ih[j][0].wait()
                ih[j][1].wait()
                pltpu.async_copy(table_hbm.at[srcb[s][j]], rows[s][j], gsem[s][j])

        def fire_scatters(s):
            for j in range(GS):
                pltpu.make_async_copy(table_hbm.at[srcb[s][j]], rows[s][j],
                                      gsem[s][j]).wait()
                pltpu.async_copy(rows[s][j], acc_sh.at[dstb[s][j]], ssem[s][j],
                                 add=True)
                if with_deg:
                    pltpu.async_copy(ones_buf.at[pl.ds(0, K)],
                                     deg_sh.at[dstb[s][j]], dsem[s][j], add=True)

        def drain_scatters(s):
            for j in range(GS):
                pltpu.make_async_copy(rows[s][j], acc_sh.at[dstb[s][j]],
                                      ssem[s][j]).wait()
                if with_deg:
                    pltpu.make_async_copy(ones_buf.at[pl.ds(0, K)],
                                          deg_sh.at[dstb[s][j]], dsem[s][j]).wait()

        fire_gathers(0, 0)
        fire_gathers(1, 1)
        fire_scatters(0)

        def outer(t, _):
            g0 = 2 * t
            drain_scatters(0)
            fire_gathers(g0, 0)
            fire_scatters(1)
            drain_scatters(1)
            fire_gathers(g0 + 1, 1)
            fire_scatters(0)
            return 0
        lax.fori_loop(1, NG // 2, outer, 0)

        fire_scatters(1)
        drain_scatters(0)
        drain_scatters(1)
        e0 = base_e + (NCH - 1) * K
        h0 = pltpu.async_copy(edge_hbm.at[pl.ds(e0, K)], srcb[0][0], isem[0][0])
        h1 = pltpu.async_copy(edge_hbm.at[pl.ds(E + e0, K)], dstb[0][0], isem[0][0])
        h0.wait()
        h1.wait()
        pltpu.async_copy(table_hbm.at[srcb[0][0]], rows[0][0], gsem[0][0]).wait()
        pltpu.async_copy(rows[0][0], acc_sh.at[dstb[0][0]], ssem[0][0],
                         add=True).wait()
        if with_deg:
            pltpu.async_copy(ones_buf.at[pl.ds(0, K)], deg_sh.at[dstb[0][0]],
                             dsem[0][0], add=True).wait()

        plsc.subcore_barrier()

        pltpu.sync_copy(acc_sh.at[pl.ds(sid * DGC, DGW)],
                        agg_out.at[cid, pl.ds(sid * DGC, DGW)])
        if with_deg:
            pltpu.sync_copy(deg_sh.at[pl.ds(sid * DGC, DGW)], zdeg_buf)
            pltpu.sync_copy(zdeg_buf,
                            deg_out.at[pl.ds(cid * N + sid * DGC, DGW)])

    mesh = plsc.VectorSubcoreMesh(core_axis_name="c", subcore_axis_name="s",
                                  num_cores=NC, num_subcores=NS)
    return pl.kernel(
        body,
        out_type=tuple(out_type),
        mesh=mesh,
        scratch_types=tuple(scratch),
        name="edge_scatter_deg" if with_deg else "edge_scatter",
    )


BLK = 1000


def _tc_mid_body(x_ref, agg_ref, deg_ref, w1_ref, b1_ref, wmu_ref, wlv_ref,
                 y_ref):
    i = pl.program_id(0)
    xb = x_ref[...] + agg_ref[0] + agg_ref[1]
    h = jnp.dot(xb, w1_ref[...], preferred_element_type=_f32) + b1_ref[...]
    h = jnp.maximum(h, 0.0)
    d0 = deg_ref[pl.ds(i, 1)]
    d1 = deg_ref[pl.ds(N // BLK + i, 1)]
    dinv = lax.rsqrt(d0[0] + d1[0] + 1.0)[:, None]
    y_ref[:, :OUT] = jnp.dot(h, wmu_ref[...], preferred_element_type=_f32) * dinv
    y_ref[:, OUT:] = jnp.dot(h, wlv_ref[...], preferred_element_type=_f32) * dinv


def _tc_mid(x, agg1, degp, W1, b1, Wmu, Wlv):
    grid = N // BLK
    return pl.pallas_call(
        _tc_mid_body,
        grid=(grid,),
        in_specs=[
            pl.BlockSpec((BLK, IN_C), lambda i: (i, 0)),
            pl.BlockSpec((NC, BLK, IN_C), lambda i: (0, i, 0)),
            pl.BlockSpec((NC * N // BLK, BLK), lambda i: (0, 0)),
            pl.BlockSpec((IN_C, HID), lambda i: (0, 0)),
            pl.BlockSpec((1, HID), lambda i: (0, 0)),
            pl.BlockSpec((HID, OUT), lambda i: (0, 0)),
            pl.BlockSpec((HID, OUT), lambda i: (0, 0)),
        ],
        out_specs=pl.BlockSpec((BLK, 2 * OUT), lambda i: (i, 0)),
        out_shape=jax.ShapeDtypeStruct((N, 2 * OUT), _f32),
        name="gcn_mid",
    )(x, agg1, degp, W1, b1, Wmu, Wlv)


def _tc_out_body(agg_ref, y_ref, deg_ref, bmu_ref, blv_ref, mu_ref, lv_ref):
    i = pl.program_id(0)
    acc = agg_ref[0] + agg_ref[1] + y_ref[...]
    d0 = deg_ref[pl.ds(i, 1)]
    d1 = deg_ref[pl.ds(N // BLK + i, 1)]
    dinv = lax.rsqrt(d0[0] + d1[0] + 1.0)
    outc = acc * dinv[:, None]
    mu_ref[...] = jnp.maximum(outc[:, :OUT] + bmu_ref[...], 0.0)
    lv_ref[...] = outc[:, OUT:] + blv_ref[...]


def _tc_out(agg2, y, degp, bmu, blv):
    grid = N // BLK
    return pl.pallas_call(
        _tc_out_body,
        grid=(grid,),
        in_specs=[
            pl.BlockSpec((NC, BLK, 2 * OUT), lambda i: (0, i, 0)),
            pl.BlockSpec((BLK, 2 * OUT), lambda i: (i, 0)),
            pl.BlockSpec((NC * N // BLK, BLK), lambda i: (0, 0)),
            pl.BlockSpec((1, OUT), lambda i: (0, 0)),
            pl.BlockSpec((1, OUT), lambda i: (0, 0)),
        ],
        out_specs=[
            pl.BlockSpec((BLK, OUT), lambda i: (i, 0)),
            pl.BlockSpec((BLK, OUT), lambda i: (i, 0)),
        ],
        out_shape=[
            jax.ShapeDtypeStruct((N, OUT), _f32),
            jax.ShapeDtypeStruct((N, OUT), _f32),
        ],
        name="gcn_out",
    )(agg2, y, degp, bmu, blv)


def kernel(x, edge_index, W1, b1, Wmu, bmu, Wlv, blv):
    ei_flat = edge_index.reshape(2 * E)
    agg1, degp_flat = _sc_scatter_pass(True)(x, ei_flat)
    degp = degp_flat.reshape(NC * N // BLK, BLK)
    y = _tc_mid(x, agg1, degp, W1, b1.reshape(1, HID), Wmu, Wlv)
    (agg2,) = _sc_scatter_pass(False)(y, ei_flat)
    mu, lv = _tc_out(agg2, y, degp,
                     bmu.reshape(1, OUT), blv.reshape(1, OUT))
    return mu, lv


# speedup vs baseline: 1.0165x; 1.0165x over previous
import functools

import jax
import jax.numpy as jnp
from jax import lax
from jax.experimental import pallas as pl
from jax.experimental.pallas import tpu as pltpu
from jax.experimental.pallas import tpu_sc as plsc

N = 10000
E = 320000
IN_C = 128
HID = 64
OUT = 64

NC = 2
NS = 16
NW = NC * NS
EPW = E // NW
K = 80
GS = 2
NCH = EPW // K
NG = (NCH - 1) // GS

DGC = 624
DGW = 640
ZR = 32
ONES = 80

_f32 = jnp.float32


@functools.cache
def _sc_scatter_pass(with_deg: bool):
    out_type = [jax.ShapeDtypeStruct((NC, N, IN_C), _f32)]
    if with_deg:
        out_type.append(jax.ShapeDtypeStruct((NC * N,), _f32))

    scratch = [
        pltpu.VMEM_SHARED((N, IN_C), _f32),
        pltpu.VMEM((ZR, IN_C), _f32),
        [[pltpu.VMEM((K, IN_C), _f32) for _ in range(GS)] for _ in range(2)],
        [[pltpu.VMEM((K,), jnp.int32) for _ in range(GS)] for _ in range(2)],
        [[pltpu.VMEM((K,), jnp.int32) for _ in range(GS)] for _ in range(2)],
        [[pltpu.SemaphoreType.DMA for _ in range(GS)] for _ in range(2)],
        [[pltpu.SemaphoreType.DMA for _ in range(GS)] for _ in range(2)],
        [[pltpu.SemaphoreType.DMA for _ in range(GS)] for _ in range(2)],
    ]
    if with_deg:
        scratch += [
            pltpu.VMEM_SHARED((N,), _f32),
            pltpu.VMEM((DGW,), _f32),
            pltpu.VMEM((ONES,), _f32),
            [[pltpu.SemaphoreType.DMA for _ in range(GS)] for _ in range(2)],
        ]

    def body(table_hbm, edge_hbm, *rest):
        if with_deg:
            (agg_out, deg_out,
             acc_sh, zero_buf, rows, srcb, dstb, isem, gsem, ssem,
             deg_sh, zdeg_buf, ones_buf, dsem) = rest
        else:
            (agg_out,
             acc_sh, zero_buf, rows, srcb, dstb, isem, gsem, ssem) = rest
            deg_out = deg_sh = zdeg_buf = ones_buf = dsem = None

        cid = lax.axis_index("c")
        sid = lax.axis_index("s")
        base_e = (cid * NS + sid) * EPW

        def fire_gathers(g, s):
            ih = []
            for j in range(GS):
                e0 = base_e + (g * GS + j) * K
                ih.append((
                    pltpu.async_copy(edge_hbm.at[pl.ds(e0, K)], srcb[s][j], isem[s][j]),
                    pltpu.async_copy(edge_hbm.at[pl.ds(E + e0, K)], dstb[s][j], isem[s][j]),
                ))
            for j in range(GS):
                ih[j][0].wait()
                ih[j][1].wait()
                pltpu.async_copy(table_hbm.at[srcb[s][j]], rows[s][j], gsem[s][j])

        def fire_scatters(s):
            for j in range(GS):
                pltpu.make_async_copy(table_hbm.at[srcb[s][j]], rows[s][j],
                                      gsem[s][j]).wait()
                pltpu.async_copy(rows[s][j], acc_sh.at[dstb[s][j]], ssem[s][j],
                                 add=True)
                if with_deg:
                    pltpu.async_copy(ones_buf.at[pl.ds(0, K)],
                                     deg_sh.at[dstb[s][j]], dsem[s][j], add=True)

        def drain_scatters(s):
            for j in range(GS):
                pltpu.make_async_copy(rows[s][j], acc_sh.at[dstb[s][j]],
                                      ssem[s][j]).wait()
                if with_deg:
                    pltpu.make_async_copy(ones_buf.at[pl.ds(0, K)],
                                          deg_sh.at[dstb[s][j]], dsem[s][j]).wait()

        fire_gathers(0, 0)
        fire_gathers(1, 1)

        zv = jnp.zeros((16,), _f32)

        def zfill(i, _):
            zero_buf[i // (IN_C // 16), pl.ds((i % (IN_C // 16)) * 16, 16)] = zv
            return 0
        lax.fori_loop(0, ZR * (IN_C // 16), zfill, 0)
        if with_deg:
            def zfill_d(i, _):
                zdeg_buf[pl.ds(i * 16, 16)] = zv
                return 0
            lax.fori_loop(0, DGW // 16, zfill_d, 0)
            ov = jnp.ones((16,), _f32)
            for j in range(ONES // 16):
                ones_buf[pl.ds(j * 16, 16)] = ov

        zh = [pltpu.async_copy(zero_buf,
                               acc_sh.at[pl.ds(sid * DGC + j * ZR, ZR)],
                               ssem[0][0])
              for j in range(DGW // ZR)]
        if with_deg:
            zh.append(pltpu.async_copy(zdeg_buf,
                                       deg_sh.at[pl.ds(sid * DGC, DGW)],
                                       ssem[0][1]))
        for h in zh:
            h.wait()
        plsc.subcore_barrier()

        fire_scatters(0)

        def outer(t, _):
            g0 = 2 * t
            drain_scatters(0)
            fire_gathers(g0, 0)
            fire_scatters(1)
            drain_scatters(1)
            fire_gathers(g0 + 1, 1)
            fire_scatters(0)
            return 0
        lax.fori_loop(1, NG // 2, outer, 0)

        fire_scatters(1)
        drain_scatters(0)
        drain_scatters(1)
        e0 = base_e + (NCH - 1) * K
        h0 = pltpu.async_copy(edge_hbm.at[pl.ds(e0, K)], srcb[0][0], isem[0][0])
        h1 = pltpu.async_copy(edge_hbm.at[pl.ds(E + e0, K)], dstb[0][0], isem[0][0])
        h0.wait()
        h1.wait()
        pltpu.async_copy(table_hbm.at[srcb[0][0]], rows[0][0], gsem[0][0]).wait()
        pltpu.async_copy(rows[0][0], acc_sh.at[dstb[0][0]], ssem[0][0],
                         add=True).wait()
        if with_deg:
            pltpu.async_copy(ones_buf.at[pl.ds(0, K)], deg_sh.at[dstb[0][0]],
                             dsem[0][0], add=True).wait()

        plsc.subcore_barrier()

        pltpu.sync_copy(acc_sh.at[pl.ds(sid * DGC, DGW)],
                        agg_out.at[cid, pl.ds(sid * DGC, DGW)])
        if with_deg:
            pltpu.sync_copy(deg_sh.at[pl.ds(sid * DGC, DGW)], zdeg_buf)
            pltpu.sync_copy(zdeg_buf,
                            deg_out.at[pl.ds(cid * N + sid * DGC, DGW)])

    mesh = plsc.VectorSubcoreMesh(core_axis_name="c", subcore_axis_name="s",
                                  num_cores=NC, num_subcores=NS)
    return pl.kernel(
        body,
        out_type=tuple(out_type),
        mesh=mesh,
        scratch_types=tuple(scratch),
        name="edge_scatter_deg" if with_deg else "edge_scatter",
    )


BLK = 1000


def _tc_mid_body(x_ref, agg_ref, deg_ref, w1_ref, b1_ref, wc_ref, y_ref):
    xb = x_ref[...] + agg_ref[0] + agg_ref[1]
    h = jnp.dot(xb, w1_ref[...], preferred_element_type=_f32) + b1_ref[...]
    h = jnp.maximum(h, 0.0)
    dinv = lax.rsqrt(deg_ref[0, 0] + deg_ref[0, 1] + 1.0)
    y = jnp.dot(h, wc_ref[...], preferred_element_type=_f32)
    y_ref[...] = y * dinv[:, None]


def _tc_mid(x, agg1, degp, W1, b1, Wc):
    grid = N // BLK
    return pl.pallas_call(
        _tc_mid_body,
        grid=(grid,),
        in_specs=[
            pl.BlockSpec((BLK, IN_C), lambda i: (i, 0)),
            pl.BlockSpec((NC, BLK, IN_C), lambda i: (0, i, 0)),
            pl.BlockSpec((1, NC, BLK), lambda i: (i, 0, 0)),
            pl.BlockSpec((IN_C, HID), lambda i: (0, 0)),
            pl.BlockSpec((1, HID), lambda i: (0, 0)),
            pl.BlockSpec((HID, 2 * OUT), lambda i: (0, 0)),
        ],
        out_specs=pl.BlockSpec((BLK, 2 * OUT), lambda i: (i, 0)),
        out_shape=jax.ShapeDtypeStruct((N, 2 * OUT), _f32),
        name="gcn_mid",
    )(x, agg1, degp, W1, b1, Wc)


def _tc_out_body(agg_ref, y_ref, deg_ref, bmu_ref, blv_ref, mu_ref, lv_ref):
    acc = agg_ref[0] + agg_ref[1] + y_ref[...]
    dinv = lax.rsqrt(deg_ref[0, 0] + deg_ref[0, 1] + 1.0)
    outc = acc * dinv[:, None]
    mu_ref[...] = jnp.maximum(outc[:, :OUT] + bmu_ref[...], 0.0)
    lv_ref[...] = outc[:, OUT:] + blv_ref[...]


def _tc_out(agg2, y, degp, bmu, blv):
    grid = N // BLK
    return pl.pallas_call(
        _tc_out_body,
        grid=(grid,),
        in_specs=[
            pl.BlockSpec((NC, BLK, 2 * OUT), lambda i: (0, i, 0)),
            pl.BlockSpec((BLK, 2 * OUT), lambda i: (i, 0)),
            pl.BlockSpec((1, NC, BLK), lambda i: (i, 0, 0)),
            pl.BlockSpec((1, OUT), lambda i: (0, 0)),
            pl.BlockSpec((1, OUT), lambda i: (0, 0)),
        ],
        out_specs=[
            pl.BlockSpec((BLK, OUT), lambda i: (i, 0)),
            pl.BlockSpec((BLK, OUT), lambda i: (i, 0)),
        ],
        out_shape=[
            jax.ShapeDtypeStruct((N, OUT), _f32),
            jax.ShapeDtypeStruct((N, OUT), _f32),
        ],
        name="gcn_out",
    )(agg2, y, degp, bmu, blv)


def kernel(x, edge_index, W1, b1, Wmu, bmu, Wlv, blv):
    ei_flat = edge_index.reshape(2 * E)
    agg1, degp_flat = _sc_scatter_pass(True)(x, ei_flat)
    degp = degp_flat.reshape(NC, N // BLK, BLK).transpose(1, 0, 2)
    Wc = jnp.concatenate([Wmu, Wlv], axis=1)
    y = _tc_mid(x, agg1, degp, W1, b1.reshape(1, HID), Wc)
    (agg2,) = _sc_scatter_pass(False)(y, ei_flat)
    mu, lv = _tc_out(agg2, y, degp,
                     bmu.reshape(1, OUT), blv.reshape(1, OUT))
    return mu, lv


# speedup vs baseline: 1.0182x; 1.0016x over previous
import functools

import jax
import jax.numpy as jnp
from jax import lax
from jax.experimental import pallas as pl
from jax.experimental.pallas import tpu as pltpu
from jax.experimental.pallas import tpu_sc as plsc

N = 10000
E = 320000
IN_C = 128
HID = 64
OUT = 64

NC = 2
NS = 16
NW = NC * NS
EPW = E // NW
K = 80
GS = 2
NCH = EPW // K
NG = (NCH - 1) // GS

DGC = 624
DGW = 640
ZR = 32
ONES = 80

_f32 = jnp.float32


@functools.cache
def _sc_scatter_pass(with_deg: bool):
    out_type = [jax.ShapeDtypeStruct((NC, N, IN_C), _f32)]
    if with_deg:
        out_type.append(jax.ShapeDtypeStruct((NC * N,), _f32))

    scratch = [
        pltpu.VMEM_SHARED((N, IN_C), _f32),
        pltpu.VMEM((ZR, IN_C), _f32),
        [[pltpu.VMEM((K, IN_C), _f32) for _ in range(GS)] for _ in range(2)],
        [[pltpu.VMEM((K,), jnp.int32) for _ in range(GS)] for _ in range(2)],
        [[pltpu.VMEM((K,), jnp.int32) for _ in range(GS)] for _ in range(2)],
        [[pltpu.SemaphoreType.DMA for _ in range(GS)] for _ in range(2)],
        [[pltpu.SemaphoreType.DMA for _ in range(GS)] for _ in range(2)],
        [[pltpu.SemaphoreType.DMA for _ in range(GS)] for _ in range(2)],
    ]
    if with_deg:
        scratch += [
            pltpu.VMEM_SHARED((N,), _f32),
            pltpu.VMEM((DGW,), _f32),
            pltpu.VMEM((ONES,), _f32),
            [[pltpu.SemaphoreType.DMA for _ in range(GS)] for _ in range(2)],
        ]

    def body(table_hbm, edge_hbm, *rest):
        if with_deg:
            (agg_out, deg_out,
             acc_sh, zero_buf, rows, srcb, dstb, isem, gsem, ssem,
             deg_sh, zdeg_buf, ones_buf, dsem) = rest
        else:
            (agg_out,
             acc_sh, zero_buf, rows, srcb, dstb, isem, gsem, ssem) = rest
            deg_out = deg_sh = zdeg_buf = ones_buf = dsem = None

        cid = lax.axis_index("c")
        sid = lax.axis_index("s")
        base_e = (cid * NS + sid) * EPW

        def fire_gathers(g, s):
            ih = []
            for j in range(GS):
                e0 = base_e + (g * GS + j) * K
                ih.append((
                    pltpu.async_copy(edge_hbm.at[pl.ds(e0, K)], srcb[s][j], isem[s][j]),
                    pltpu.async_copy(edge_hbm.at[pl.ds(E + e0, K)], dstb[s][j], isem[s][j]),
                ))
            for j in range(GS):
                ih[j][0].wait()
                ih[j][1].wait()
                pltpu.async_copy(table_hbm.at[srcb[s][j]], rows[s][j], gsem[s][j])

        def fire_scatters(s):
            for j in range(GS):
                pltpu.make_async_copy(table_hbm.at[srcb[s][j]], rows[s][j],
                                      gsem[s][j]).wait()
                pltpu.async_copy(rows[s][j], acc_sh.at[dstb[s][j]], ssem[s][j],
                                 add=True)
                if with_deg:
                    pltpu.async_copy(ones_buf.at[pl.ds(0, K)],
                                     deg_sh.at[dstb[s][j]], dsem[s][j], add=True)

        def drain_scatters(s):
            for j in range(GS):
                pltpu.make_async_copy(rows[s][j], acc_sh.at[dstb[s][j]],
                                      ssem[s][j]).wait()
                if with_deg:
                    pltpu.make_async_copy(ones_buf.at[pl.ds(0, K)],
                                          deg_sh.at[dstb[s][j]], dsem[s][j]).wait()

        fire_gathers(0, 0)
        fire_gathers(1, 1)

        zv = jnp.zeros((16,), _f32)

        def zfill(i, _):
            zero_buf[i // (IN_C // 16), pl.ds((i % (IN_C // 16)) * 16, 16)] = zv
            return 0
        lax.fori_loop(0, ZR * (IN_C // 16), zfill, 0)
        if with_deg:
            def zfill_d(i, _):
                zdeg_buf[pl.ds(i * 16, 16)] = zv
                return 0
            lax.fori_loop(0, DGW // 16, zfill_d, 0)
            ov = jnp.ones((16,), _f32)
            for j in range(ONES // 16):
                ones_buf[pl.ds(j * 16, 16)] = ov

        zh = [pltpu.async_copy(zero_buf,
                               acc_sh.at[pl.ds(sid * DGC + j * ZR, ZR)],
                               ssem[0][0])
              for j in range(DGW // ZR)]
        if with_deg:
            zh.append(pltpu.async_copy(zdeg_buf,
                                       deg_sh.at[pl.ds(sid * DGC, DGW)],
                                       ssem[0][1]))
        for h in zh:
            h.wait()
        plsc.subcore_barrier()

        fire_scatters(0)

        def outer(t, _):
            g0 = 2 * t
            drain_scatters(0)
            fire_gathers(g0, 0)
            fire_scatters(1)
            drain_scatters(1)
            fire_gathers(g0 + 1, 1)
            fire_scatters(0)
            return 0
        lax.fori_loop(1, NG // 2, outer, 0)

        fire_scatters(1)
        drain_scatters(0)
        drain_scatters(1)
        e0 = base_e + (NCH - 1) * K
        h0 = pltpu.async_copy(edge_hbm.at[pl.ds(e0, K)], srcb[0][0], isem[0][0])
        h1 = pltpu.async_copy(edge_hbm.at[pl.ds(E + e0, K)], dstb[0][0], isem[0][0])
        h0.wait()
        h1.wait()
        pltpu.async_copy(table_hbm.at[srcb[0][0]], rows[0][0], gsem[0][0]).wait()
        pltpu.async_copy(rows[0][0], acc_sh.at[dstb[0][0]], ssem[0][0],
                         add=True).wait()
        if with_deg:
            pltpu.async_copy(ones_buf.at[pl.ds(0, K)], deg_sh.at[dstb[0][0]],
                             dsem[0][0], add=True).wait()

        plsc.subcore_barrier()

        oh = pltpu.async_copy(acc_sh.at[pl.ds(sid * DGC, DGW)],
                              agg_out.at[cid, pl.ds(sid * DGC, DGW)],
                              gsem[0][0])
        if with_deg:
            pltpu.async_copy(deg_sh.at[pl.ds(sid * DGC, DGW)], zdeg_buf,
                             gsem[0][1]).wait()
            pltpu.async_copy(zdeg_buf,
                             deg_out.at[pl.ds(cid * N + sid * DGC, DGW)],
                             gsem[1][0]).wait()
        oh.wait()

    mesh = plsc.VectorSubcoreMesh(core_axis_name="c", subcore_axis_name="s",
                                  num_cores=NC, num_subcores=NS)
    return pl.kernel(
        body,
        out_type=tuple(out_type),
        mesh=mesh,
        scratch_types=tuple(scratch),
        name="edge_scatter_deg" if with_deg else "edge_scatter",
    )


BLK = 1000


def _tc_mid_body(x_ref, agg_ref, deg_ref, w1_ref, b1_ref, wc_ref, y_ref):
    xb = x_ref[...] + agg_ref[0] + agg_ref[1]
    h = jnp.dot(xb, w1_ref[...], preferred_element_type=_f32) + b1_ref[...]
    h = jnp.maximum(h, 0.0)
    dinv = lax.rsqrt(deg_ref[0, 0] + deg_ref[0, 1] + 1.0)
    y = jnp.dot(h, wc_ref[...], preferred_element_type=_f32)
    y_ref[...] = y * dinv[:, None]


def _tc_mid(x, agg1, degp, W1, b1, Wc):
    grid = N // BLK
    return pl.pallas_call(
        _tc_mid_body,
        grid=(grid,),
        in_specs=[
            pl.BlockSpec((BLK, IN_C), lambda i: (i, 0)),
            pl.BlockSpec((NC, BLK, IN_C), lambda i: (0, i, 0)),
            pl.BlockSpec((1, NC, BLK), lambda i: (i, 0, 0)),
            pl.BlockSpec((IN_C, HID), lambda i: (0, 0)),
            pl.BlockSpec((1, HID), lambda i: (0, 0)),
            pl.BlockSpec((HID, 2 * OUT), lambda i: (0, 0)),
        ],
        out_specs=pl.BlockSpec((BLK, 2 * OUT), lambda i: (i, 0)),
        out_shape=jax.ShapeDtypeStruct((N, 2 * OUT), _f32),
        name="gcn_mid",
    )(x, agg1, degp, W1, b1, Wc)


def _tc_out_body(agg_ref, y_ref, deg_ref, bmu_ref, blv_ref, mu_ref, lv_ref):
    acc = agg_ref[0] + agg_ref[1] + y_ref[...]
    dinv = lax.rsqrt(deg_ref[0, 0] + deg_ref[0, 1] + 1.0)
    outc = acc * dinv[:, None]
    mu_ref[...] = jnp.maximum(outc[:, :OUT] + bmu_ref[...], 0.0)
    lv_ref[...] = outc[:, OUT:] + blv_ref[...]


def _tc_out(agg2, y, degp, bmu, blv):
    grid = N // BLK
    return pl.pallas_call(
        _tc_out_body,
        grid=(grid,),
        in_specs=[
            pl.BlockSpec((NC, BLK, 2 * OUT), lambda i: (0, i, 0)),
            pl.BlockSpec((BLK, 2 * OUT), lambda i: (i, 0)),
            pl.BlockSpec((1, NC, BLK), lambda i: (i, 0, 0)),
            pl.BlockSpec((1, OUT), lambda i: (0, 0)),
            pl.BlockSpec((1, OUT), lambda i: (0, 0)),
        ],
        out_specs=[
            pl.BlockSpec((BLK, OUT), lambda i: (i, 0)),
            pl.BlockSpec((BLK, OUT), lambda i: (i, 0)),
        ],
        out_shape=[
            jax.ShapeDtypeStruct((N, OUT), _f32),
            jax.ShapeDtypeStruct((N, OUT), _f32),
        ],
        name="gcn_out",
    )(agg2, y, degp, bmu, blv)


def kernel(x, edge_index, W1, b1, Wmu, bmu, Wlv, blv):
    ei_flat = edge_index.reshape(2 * E)
    agg1, degp_flat = _sc_scatter_pass(True)(x, ei_flat)
    degp = degp_flat.reshape(NC, N // BLK, BLK).transpose(1, 0, 2)
    Wc = jnp.concatenate([Wmu, Wlv], axis=1)
    y = _tc_mid(x, agg1, degp, W1, b1.reshape(1, HID), Wc)
    (agg2,) = _sc_scatter_pass(False)(y, ei_flat)
    mu, lv = _tc_out(agg2, y, degp,
                     bmu.reshape(1, OUT), blv.reshape(1, OUT))
    return mu, lv
